# baseline (speedup 1.0000x reference)
import jax
import jax.numpy as jnp
from jax import lax
from jax.experimental import pallas as pl
from jax.experimental.pallas import tpu as pltpu
from jax.experimental.pallas import tpu_sc as plsc

NC = 2
NS = 16
NW = NC * NS
L = 16

MC = 1000000
DD = 64
BB = 65536
GROUP = 512
NT_FULL = MC // GROUP
PART_BASE = NT_FULL * GROUP
PART_W = MC - PART_BASE
PART_OWNER = NT_FULL % NW
PART_BIN = NT_FULL // NW
OW_SHIFT = 9
BIN_SHIFT = 14
SEC = 8192
NSEC = BB // SEC
CAP = 6144
PRE = 64
NBINS = 64


def _i32(x):
    return jnp.full((L,), x, jnp.int32)


def _sc_body(memT, idx_hbm, val_hbm, outT,
             idx_buf, tgt_c, pos_c, binned_t, binned_p,
             runhist, binbase, nextfree, chunk, chunk2, chunk64, vbuf,
             st16, clbuf, st_a, st_b, gbuf_a, gbuf_b,
             sem_i, sem_o, sem_i2, sem_o2, sem_g, sem_ga, sem_gb):
    wid = lax.axis_index("s") * NC + lax.axis_index("c")
    iota = lax.iota(jnp.int32, L)
    widv = _i32(wid)

    def sec_body(s, cv):
        pltpu.sync_copy(idx_hbm.at[pl.ds(s * SEC, SEC)], idx_buf)

        def scan_body(i, cv):
            for q in range(4):
                off = i * 4 * L + q * L
                v = idx_buf[pl.ds(off, L)]
                m = (lax.shift_right_logical(v, _i32(OW_SHIFT))
                     & _i32(NW - 1)) == widv

                def compact(cvv):
                    bvec = _i32(s * SEC) + _i32(off) + iota
                    mm = m & (cvv < _i32(CAP - L))
                    mi = jnp.where(mm, _i32(1), _i32(0))
                    dest = cvv + plsc.cumsum(mi) - mi
                    plsc.store_scatter(tgt_c, [dest], v, mask=mm)
                    plsc.store_scatter(pos_c, [dest], bvec, mask=mm)
                    return cvv + plsc.all_reduce_population_count(mm)

                cv = lax.cond(jnp.any(m), compact, lambda cvv: cvv, cv)
            return cv

        return lax.fori_loop(0, SEC // (4 * L), scan_body, cv)

    cnt_vec = lax.fori_loop(0, NSEC, sec_body, _i32(0))
    cnt = jnp.max(cnt_vec)
    nv = (cnt + L - 1) // L

    def zb(j, _):
        runhist[pl.ds(j * L, L)] = _i32(0)
        return 0

    lax.fori_loop(0, NBINS // L, zb, 0)

    def hist_body(i, _):
        base = i * L
        t = tgt_c[pl.ds(base, L)]
        valid = (_i32(base) + iota) < _i32(cnt)
        binv = jnp.clip(lax.shift_right_logical(t, _i32(BIN_SHIFT)), 0,
                        NBINS - 1)
        rc, lastm = plsc.scan_count(binv, valid)
        plsc.addupdate_scatter(runhist, [binv], rc, mask=lastm & valid)
        return 0

    lax.fori_loop(0, nv, hist_body, 0)

    def scan_bins(j, run):
        h = runhist[pl.ds(j * L, L)]
        c = plsc.cumsum(h)
        binbase[pl.ds(j * L, L)] = _i32(run) + c - h
        return run + jnp.sum(h)

    lax.fori_loop(0, NBINS // L, scan_bins, jnp.int32(0))

    def cpnf(j, _):
        nextfree[pl.ds(j * L, L)] = _i32(0)
        return 0

    lax.fori_loop(0, NBINS // L, cpnf, 0)

    def place_body(i, _):
        base = i * L
        t = tgt_c[pl.ds(base, L)]
        p = pos_c[pl.ds(base, L)]
        valid = (_i32(base) + iota) < _i32(cnt)
        binv = jnp.clip(lax.shift_right_logical(t, _i32(BIN_SHIFT)), 0,
                        NBINS - 1)
        rc, lastm = plsc.scan_count(binv, valid)
        run = plsc.load_gather(nextfree, [binv], mask=valid)
        bb = plsc.load_gather(binbase, [binv], mask=valid)
        dest = bb + run + rc - _i32(1)
        plsc.store_scatter(binned_t, [dest], t, mask=valid)
        plsc.store_scatter(binned_p, [dest], p, mask=valid)
        plsc.addupdate_scatter(nextfree, [binv], rc, mask=lastm & valid)
        return 0

    lax.fori_loop(0, nv, place_body, 0)

    def bin_bounds(k):
        hb = (k // L) * L
        hv = runhist[pl.ds(hb, L)]
        bv = binbase[pl.ds(hb, L)]
        selk = iota == _i32(k - hb)
        n_k = jnp.sum(jnp.where(selk, hv, _i32(0)))
        b_k = jnp.sum(jnp.where(selk, bv, _i32(0)))
        return n_k, b_k

    def prefetch_gather(k, stbuf, gbuf, semx):
        n_k, b_k = bin_bounds(k)
        nm1 = jnp.maximum(n_k - 1, 0)
        for q in range(PRE // L):
            li = _i32(b_k) + jnp.minimum(_i32(q * L) + iota, _i32(nm1))
            pb = plsc.load_gather(binned_p, [li])
            stbuf[pl.ds(q * L, L)] = jnp.clip(pb, 0, BB - 1)
        pltpu.async_copy(val_hbm.at[stbuf], gbuf, semx)

    def apply_updates(k, cbuf, stbuf, gbuf, semx):
        n_k, b_k = bin_bounds(k)
        pltpu.make_async_copy(val_hbm.at[stbuf], gbuf, semx).wait()
        nmain = jnp.minimum(n_k, PRE)

        def ent_body4(i4, _3):
            for u in range(4):
                i = i4 * 4 + u
                clv = (plsc.load_gather(binned_t, [_i32(b_k) + _i32(i)])
                       & _i32(GROUP - 1))
                for kq in range(DD // L):
                    x = gbuf[i, pl.ds(kq * L, L)]
                    plsc.store_scatter(cbuf, [iota + _i32(kq * L), clv], x)
            return 0

        lax.fori_loop(0, nmain // 4, ent_body4, 0)

        def ent_body(i, _3):
            clv = (plsc.load_gather(binned_t, [_i32(b_k) + _i32(i)])
                   & _i32(GROUP - 1))
            for kq in range(DD // L):
                x = gbuf[i, pl.ds(kq * L, L)]
                plsc.store_scatter(cbuf, [iota + _i32(kq * L), clv], x)
            return 0

        lax.fori_loop((nmain // 4) * 4, nmain, ent_body, 0)

        nrounds = (n_k + L - 1) // L

        def round_body(r, _2):
            off = b_k + r * L
            li = _i32(off) + jnp.minimum(iota, _i32(n_k - r * L - 1))
            col16 = plsc.load_gather(binned_t, [li])
            pos16 = plsc.load_gather(binned_p, [li])
            st16[pl.ds(0, L)] = pos16
            pltpu.async_copy(val_hbm.at[st16], vbuf, sem_g).wait()
            clbuf[pl.ds(0, L)] = col16 & _i32(GROUP - 1)
            m_sc = jnp.minimum(n_k - r * L, L)

            def ent2(i, _3):
                clv = plsc.load_gather(clbuf, [_i32(i)])
                for kq in range(DD // L):
                    x = vbuf[i, pl.ds(kq * L, L)]
                    plsc.store_scatter(
                        cbuf, [iota + _i32(kq * L), clv], x)
                return 0

            lax.fori_loop(0, m_sc, ent2, 0)
            return 0

        lax.fori_loop(PRE // L, nrounds, round_body, 0)

    my_nt = (NT_FULL - wid + NW - 1) // NW

    def start_in(k, cbuf, sem):
        t = wid + k * NW
        pltpu.async_copy(memT.at[:, pl.ds(t * GROUP, GROUP)], cbuf, sem)

    def wait_in(cbuf, sem):
        pltpu.make_async_copy(memT.at[:, pl.ds(0, GROUP)], cbuf, sem).wait()

    def start_out(k, cbuf, sem):
        t = wid + k * NW
        pltpu.async_copy(cbuf, outT.at[:, pl.ds(t * GROUP, GROUP)], sem)

    def wait_out(cbuf, sem):
        pltpu.make_async_copy(cbuf, outT.at[:, pl.ds(0, GROUP)], sem).wait()

    def when(cond, fn):
        def b(_, __):
            fn()
            return 0

        lax.fori_loop(0, jnp.where(cond, 1, 0), b, 0)

    start_in(0, chunk, sem_i)
    npairs = (my_nt + 1) // 2

    def pair_body(j, _):
        k0 = j * 2
        k1 = k0 + 1
        when((j > 0) & (k1 < my_nt), lambda: wait_out(chunk2, sem_o2))
        when(k1 < my_nt, lambda: start_in(k1, chunk2, sem_i2))
        prefetch_gather(k0, st_a, gbuf_a, sem_ga)
        when(k1 < my_nt, lambda: prefetch_gather(k1, st_b, gbuf_b, sem_gb))
        wait_in(chunk, sem_i)
        apply_updates(k0, chunk, st_a, gbuf_a, sem_ga)
        start_out(k0, chunk, sem_o)

        def do_b():
            wait_in(chunk2, sem_i2)
            apply_updates(k1, chunk2, st_b, gbuf_b, sem_gb)
            start_out(k1, chunk2, sem_o2)

        when(k1 < my_nt, do_b)

        def prefetch_a():
            wait_out(chunk, sem_o)
            start_in(k0 + 2, chunk, sem_i)

        when(k0 + 2 < my_nt, prefetch_a)
        return 0

    lax.fori_loop(0, npairs, pair_body, 0)
    wait_out(chunk, sem_o)
    when(my_nt >= 2, lambda: wait_out(chunk2, sem_o2))

    def part_body(_, _2):
        prefetch_gather(PART_BIN, st_a, gbuf_a, sem_ga)
        pltpu.async_copy(memT.at[:, pl.ds(PART_BASE, PART_W)], chunk64,
                         sem_i).wait()
        apply_updates(PART_BIN, chunk64, st_a, gbuf_a, sem_ga)
        pltpu.async_copy(chunk64, outT.at[:, pl.ds(PART_BASE, PART_W)],
                         sem_o).wait()
        return 0

    is_owner = jnp.where(wid == PART_OWNER, 1, 0)
    lax.fori_loop(0, is_owner, part_body, 0)


def _make_sc_update():
    mesh = plsc.VectorSubcoreMesh(core_axis_name="c", subcore_axis_name="s")
    return pl.kernel(
        _sc_body,
        out_type=jax.ShapeDtypeStruct((DD, MC), jnp.float32),
        mesh=mesh,
        compiler_params=pltpu.CompilerParams(needs_layout_passes=False),
        scratch_types=[
            pltpu.VMEM((SEC,), jnp.int32),
            pltpu.VMEM((CAP,), jnp.int32),
            pltpu.VMEM((CAP,), jnp.int32),
            pltpu.VMEM((CAP,), jnp.int32),
            pltpu.VMEM((CAP,), jnp.int32),
            pltpu.VMEM((NBINS,), jnp.int32),
            pltpu.VMEM((NBINS,), jnp.int32),
            pltpu.VMEM((NBINS,), jnp.int32),
            pltpu.VMEM((DD, GROUP), jnp.float32),
            pltpu.VMEM((DD, GROUP), jnp.float32),
            pltpu.VMEM((DD, PART_W), jnp.float32),
            pltpu.VMEM((L, 128), jnp.float32),
            pltpu.VMEM((L,), jnp.int32),
            pltpu.VMEM((L,), jnp.int32),
            pltpu.VMEM((PRE,), jnp.int32),
            pltpu.VMEM((PRE,), jnp.int32),
            pltpu.VMEM((PRE, 128), jnp.float32),
            pltpu.VMEM((PRE, 128), jnp.float32),
            pltpu.SemaphoreType.DMA,
            pltpu.SemaphoreType.DMA,
            pltpu.SemaphoreType.DMA,
            pltpu.SemaphoreType.DMA,
            pltpu.SemaphoreType.DMA,
            pltpu.SemaphoreType.DMA,
            pltpu.SemaphoreType.DMA,
        ],
    )


@jax.jit
def kernel(mem, idx, val):
    idx32 = idx.astype(jnp.int32)
    memT = mem.T
    val128 = jnp.pad(val, ((0, 0), (0, 128 - DD)))
    outT = _make_sc_update()(memT, idx32, val128)
    return outT.T


# speedup vs baseline: 1.1374x; 1.1374x over previous
import jax
import jax.numpy as jnp
from jax import lax
from jax.experimental import pallas as pl
from jax.experimental.pallas import tpu as pltpu
from jax.experimental.pallas import tpu_sc as plsc

NC = 2
NS = 16
NW = NC * NS
L = 16

MC = 1000000
DD = 64
BB = 65536
GROUP = 512
NT_FULL = MC // GROUP
PART_BASE = NT_FULL * GROUP
PART_W = MC - PART_BASE
PART_OWNER = NT_FULL % NW
PART_BIN = NT_FULL // NW
OW_SHIFT = 9
BIN_SHIFT = 14
SEC = 8192
NSEC = BB // SEC
CAP = 6144
PRE = 64
NBINS = 64


def _i32(x):
    return jnp.full((L,), x, jnp.int32)


def _sc_body(memT, idx_hbm, val_hbm, outT,
             idx_buf, tgt_c, pos_c, binned_t, binned_p,
             runhist, binbase, nextfree, chunk, chunk2, chunk64, vbuf,
             st16, clbuf, st_a, st_b, gbuf_a, gbuf_b,
             sem_i, sem_o, sem_i2, sem_o2, sem_g, sem_ga, sem_gb):
    wid = lax.axis_index("s") * NC + lax.axis_index("c")
    iota = lax.iota(jnp.int32, L)
    widv = _i32(wid)

    def sec_body(s, cv):
        pltpu.sync_copy(idx_hbm.at[pl.ds(s * SEC, SEC)], idx_buf)

        def scan_body(i, cv):
            for q in range(4):
                off = i * 4 * L + q * L
                v = idx_buf[pl.ds(off, L)]
                bvec = _i32(s * SEC) + _i32(off) + iota
                m = (lax.shift_right_logical(v, _i32(OW_SHIFT))
                     & _i32(NW - 1)) == widv
                m = m & (cv < _i32(CAP - L))
                mi = jnp.where(m, _i32(1), _i32(0))
                dest = cv + plsc.cumsum(mi) - mi
                plsc.store_scatter(tgt_c, [dest], v, mask=m)
                plsc.store_scatter(pos_c, [dest], bvec, mask=m)
                cv = cv + plsc.all_reduce_population_count(m)
            return cv

        return lax.fori_loop(0, SEC // (4 * L), scan_body, cv)

    cnt_vec = lax.fori_loop(0, NSEC, sec_body, _i32(0))
    cnt = jnp.max(cnt_vec)
    nv = (cnt + L - 1) // L

    def zb(j, _):
        runhist[pl.ds(j * L, L)] = _i32(0)
        return 0

    lax.fori_loop(0, NBINS // L, zb, 0)

    def hist_body(i, _):
        base = i * L
        t = tgt_c[pl.ds(base, L)]
        valid = (_i32(base) + iota) < _i32(cnt)
        binv = jnp.clip(lax.shift_right_logical(t, _i32(BIN_SHIFT)), 0,
                        NBINS - 1)
        rc, lastm = plsc.scan_count(binv, valid)
        plsc.addupdate_scatter(runhist, [binv], rc, mask=lastm & valid)
        return 0

    lax.fori_loop(0, nv, hist_body, 0)

    def scan_bins(j, run):
        h = runhist[pl.ds(j * L, L)]
        c = plsc.cumsum(h)
        binbase[pl.ds(j * L, L)] = _i32(run) + c - h
        return run + jnp.sum(h)

    lax.fori_loop(0, NBINS // L, scan_bins, jnp.int32(0))

    def cpnf(j, _):
        nextfree[pl.ds(j * L, L)] = _i32(0)
        return 0

    lax.fori_loop(0, NBINS // L, cpnf, 0)

    def place_body(i, _):
        base = i * L
        t = tgt_c[pl.ds(base, L)]
        p = pos_c[pl.ds(base, L)]
        valid = (_i32(base) + iota) < _i32(cnt)
        binv = jnp.clip(lax.shift_right_logical(t, _i32(BIN_SHIFT)), 0,
                        NBINS - 1)
        rc, lastm = plsc.scan_count(binv, valid)
        run = plsc.load_gather(nextfree, [binv], mask=valid)
        bb = plsc.load_gather(binbase, [binv], mask=valid)
        dest = bb + run + rc - _i32(1)
        plsc.store_scatter(binned_t, [dest], t, mask=valid)
        plsc.store_scatter(binned_p, [dest], p, mask=valid)
        plsc.addupdate_scatter(nextfree, [binv], rc, mask=lastm & valid)
        return 0

    lax.fori_loop(0, nv, place_body, 0)

    def bin_bounds(k):
        hb = (k // L) * L
        hv = runhist[pl.ds(hb, L)]
        bv = binbase[pl.ds(hb, L)]
        selk = iota == _i32(k - hb)
        n_k = jnp.sum(jnp.where(selk, hv, _i32(0)))
        b_k = jnp.sum(jnp.where(selk, bv, _i32(0)))
        return n_k, b_k

    def prefetch_gather(k, stbuf, gbuf, semx):
        n_k, b_k = bin_bounds(k)
        nm1 = jnp.maximum(n_k - 1, 0)
        for q in range(PRE // L):
            li = _i32(b_k) + jnp.minimum(_i32(q * L) + iota, _i32(nm1))
            pb = plsc.load_gather(binned_p, [li])
            stbuf[pl.ds(q * L, L)] = jnp.clip(pb, 0, BB - 1)
        pltpu.async_copy(val_hbm.at[stbuf], gbuf, semx)

    def apply_updates(k, cbuf, stbuf, gbuf, semx):
        n_k, b_k = bin_bounds(k)
        pltpu.make_async_copy(val_hbm.at[stbuf], gbuf, semx).wait()
        nmain = jnp.minimum(n_k, PRE)

        def ent_body4(i4, _3):
            for u in range(4):
                i = i4 * 4 + u
                clv = (plsc.load_gather(binned_t, [_i32(b_k) + _i32(i)])
                       & _i32(GROUP - 1))
                for kq in range(DD // L):
                    x = gbuf[i, pl.ds(kq * L, L)]
                    plsc.store_scatter(cbuf, [iota + _i32(kq * L), clv], x)
            return 0

        lax.fori_loop(0, nmain // 4, ent_body4, 0)

        def ent_body(i, _3):
            clv = (plsc.load_gather(binned_t, [_i32(b_k) + _i32(i)])
                   & _i32(GROUP - 1))
            for kq in range(DD // L):
                x = gbuf[i, pl.ds(kq * L, L)]
                plsc.store_scatter(cbuf, [iota + _i32(kq * L), clv], x)
            return 0

        lax.fori_loop((nmain // 4) * 4, nmain, ent_body, 0)

        nrounds = (n_k + L - 1) // L

        def round_body(r, _2):
            off = b_k + r * L
            li = _i32(off) + jnp.minimum(iota, _i32(n_k - r * L - 1))
            col16 = plsc.load_gather(binned_t, [li])
            pos16 = plsc.load_gather(binned_p, [li])
            st16[pl.ds(0, L)] = pos16
            pltpu.async_copy(val_hbm.at[st16], vbuf, sem_g).wait()
            clbuf[pl.ds(0, L)] = col16 & _i32(GROUP - 1)
            m_sc = jnp.minimum(n_k - r * L, L)

            def ent2(i, _3):
                clv = plsc.load_gather(clbuf, [_i32(i)])
                for kq in range(DD // L):
                    x = vbuf[i, pl.ds(kq * L, L)]
                    plsc.store_scatter(
                        cbuf, [iota + _i32(kq * L), clv], x)
                return 0

            lax.fori_loop(0, m_sc, ent2, 0)
            return 0

        lax.fori_loop(PRE // L, nrounds, round_body, 0)

    my_nt = (NT_FULL - wid + NW - 1) // NW

    def start_in(k, cbuf, sem):
        t = wid + k * NW
        pltpu.async_copy(memT.at[:, pl.ds(t * GROUP, GROUP)], cbuf, sem)

    def wait_in(cbuf, sem):
        pltpu.make_async_copy(memT.at[:, pl.ds(0, GROUP)], cbuf, sem).wait()

    def start_out(k, cbuf, sem):
        t = wid + k * NW
        pltpu.async_copy(cbuf, outT.at[:, pl.ds(t * GROUP, GROUP)], sem)

    def wait_out(cbuf, sem):
        pltpu.make_async_copy(cbuf, outT.at[:, pl.ds(0, GROUP)], sem).wait()

    def when(cond, fn):
        def b(_, __):
            fn()
            return 0

        lax.fori_loop(0, jnp.where(cond, 1, 0), b, 0)

    start_in(0, chunk, sem_i)
    npairs = (my_nt + 1) // 2

    def pair_body(j, _):
        k0 = j * 2
        k1 = k0 + 1
        when((j > 0) & (k1 < my_nt), lambda: wait_out(chunk2, sem_o2))
        when(k1 < my_nt, lambda: start_in(k1, chunk2, sem_i2))
        prefetch_gather(k0, st_a, gbuf_a, sem_ga)
        when(k1 < my_nt, lambda: prefetch_gather(k1, st_b, gbuf_b, sem_gb))
        wait_in(chunk, sem_i)
        apply_updates(k0, chunk, st_a, gbuf_a, sem_ga)
        start_out(k0, chunk, sem_o)

        def do_b():
            wait_in(chunk2, sem_i2)
            apply_updates(k1, chunk2, st_b, gbuf_b, sem_gb)
            start_out(k1, chunk2, sem_o2)

        when(k1 < my_nt, do_b)

        def prefetch_a():
            wait_out(chunk, sem_o)
            start_in(k0 + 2, chunk, sem_i)

        when(k0 + 2 < my_nt, prefetch_a)
        return 0

    lax.fori_loop(0, npairs, pair_body, 0)
    wait_out(chunk, sem_o)
    when(my_nt >= 2, lambda: wait_out(chunk2, sem_o2))

    def part_body(_, _2):
        prefetch_gather(PART_BIN, st_a, gbuf_a, sem_ga)
        pltpu.async_copy(memT.at[:, pl.ds(PART_BASE, PART_W)], chunk64,
                         sem_i).wait()
        apply_updates(PART_BIN, chunk64, st_a, gbuf_a, sem_ga)
        pltpu.async_copy(chunk64, outT.at[:, pl.ds(PART_BASE, PART_W)],
                         sem_o).wait()
        return 0

    is_owner = jnp.where(wid == PART_OWNER, 1, 0)
    lax.fori_loop(0, is_owner, part_body, 0)


def _make_sc_update():
    mesh = plsc.VectorSubcoreMesh(core_axis_name="c", subcore_axis_name="s")
    return pl.kernel(
        _sc_body,
        out_type=jax.ShapeDtypeStruct((DD, MC), jnp.float32),
        mesh=mesh,
        compiler_params=pltpu.CompilerParams(needs_layout_passes=False),
        scratch_types=[
            pltpu.VMEM((SEC,), jnp.int32),
            pltpu.VMEM((CAP,), jnp.int32),
            pltpu.VMEM((CAP,), jnp.int32),
            pltpu.VMEM((CAP,), jnp.int32),
            pltpu.VMEM((CAP,), jnp.int32),
            pltpu.VMEM((NBINS,), jnp.int32),
            pltpu.VMEM((NBINS,), jnp.int32),
            pltpu.VMEM((NBINS,), jnp.int32),
            pltpu.VMEM((DD, GROUP), jnp.float32),
            pltpu.VMEM((DD, GROUP), jnp.float32),
            pltpu.VMEM((DD, PART_W), jnp.float32),
            pltpu.VMEM((L, 128), jnp.float32),
            pltpu.VMEM((L,), jnp.int32),
            pltpu.VMEM((L,), jnp.int32),
            pltpu.VMEM((PRE,), jnp.int32),
            pltpu.VMEM((PRE,), jnp.int32),
            pltpu.VMEM((PRE, 128), jnp.float32),
            pltpu.VMEM((PRE, 128), jnp.float32),
            pltpu.SemaphoreType.DMA,
            pltpu.SemaphoreType.DMA,
            pltpu.SemaphoreType.DMA,
            pltpu.SemaphoreType.DMA,
            pltpu.SemaphoreType.DMA,
            pltpu.SemaphoreType.DMA,
            pltpu.SemaphoreType.DMA,
        ],
    )


@jax.jit
def kernel(mem, idx, val):
    idx32 = idx.astype(jnp.int32)
    memT = mem.T
    val128 = jnp.pad(val, ((0, 0), (0, 128 - DD)))
    outT = _make_sc_update()(memT, idx32, val128)
    return outT.T
